# Initial kernel scaffold; baseline (speedup 1.0000x reference)
import functools
import math

import jax
import jax.numpy as jnp
from jax import lax
from jax.experimental import pallas as pl
from jax.experimental.pallas import tpu as pltpu
from jax.experimental.pallas import tpu_sc as plsc

_D = 64
_IDX_W = 128

_info = plsc.get_sparse_core_info()
_NC, _NS = _info.num_cores, _info.num_subcores
_NW = _NC * _NS


def _sc_gather_sum(nrows: int):
    rows_per_w = nrows // _NW
    mesh = plsc.VectorSubcoreMesh(core_axis_name="c", subcore_axis_name="s")

    @functools.partial(
        pl.kernel,
        mesh=mesh,
        out_type=jax.ShapeDtypeStruct((nrows, _IDX_W, _D), jnp.float32),
        scratch_types=[
            pltpu.VMEM((_IDX_W,), jnp.int32),
            pltpu.VMEM((_IDX_W,), jnp.int32),
            pltpu.VMEM((_IDX_W,), jnp.int32),
            pltpu.VMEM((_IDX_W, _D), jnp.float32),
            pltpu.VMEM((_IDX_W, _D), jnp.float32),
            pltpu.VMEM((_IDX_W, _D), jnp.float32),
            pltpu.SemaphoreType.DMA,
            pltpu.SemaphoreType.DMA,
            pltpu.SemaphoreType.DMA,
        ],
    )
    def k(t0, t1, t2, e0, e1, e2, out, i0, i1, i2, r0, r1, r2, s0, s1, s2):
        wid = lax.axis_index("s") * _NC + lax.axis_index("c")
        row0 = wid * rows_per_w

        def chunk(c, carry):
            row = row0 + c
            pltpu.sync_copy(t0.at[row], i0)
            pltpu.sync_copy(t1.at[row], i1)
            pltpu.sync_copy(t2.at[row], i2)
            cp0 = pltpu.async_copy(e0.at[i0], r0, s0)
            cp1 = pltpu.async_copy(e1.at[i1], r1, s1)
            cp2 = pltpu.async_copy(e2.at[i2], r2, s2)
            cp0.wait()
            cp1.wait()
            cp2.wait()

            def add_row(i, carry2):
                for j in range(_D // 16):
                    sl = pl.ds(j * 16, 16)
                    r0[i, sl] = r0[i, sl] + r1[i, sl] + r2[i, sl]
                return carry2

            lax.fori_loop(0, _IDX_W, add_row, 0)
            pltpu.sync_copy(r0, out.at[row])
            return carry

        lax.fori_loop(0, rows_per_w, chunk, 0)

    return k


def _tc_mlp_add(n: int, bn: int):
    grid = n // bn
    inv_sqrt2 = 1.0 / math.sqrt(2.0)

    def body(h_ref, x_ref, w1_ref, b1_ref, w2_ref, b2_ref, o_ref):
        x = x_ref[...]
        y = jnp.dot(x, w1_ref[...], preferred_element_type=jnp.float32)
        y = y + b1_ref[...]
        y = 0.5 * y * (1.0 + lax.erf(y * inv_sqrt2))
        z = jnp.dot(y, w2_ref[...], preferred_element_type=jnp.float32)
        o_ref[...] = h_ref[...] + z + b2_ref[...]

    return pl.pallas_call(
        body,
        grid=(grid,),
        in_specs=[
            pl.BlockSpec((bn, _D), lambda i: (i, 0)),
            pl.BlockSpec((bn, _D), lambda i: (i, 0)),
            pl.BlockSpec((_D, _D), lambda i: (0, 0)),
            pl.BlockSpec((1, _D), lambda i: (0, 0)),
            pl.BlockSpec((_D, _D), lambda i: (0, 0)),
            pl.BlockSpec((1, _D), lambda i: (0, 0)),
        ],
        out_specs=pl.BlockSpec((bn, _D), lambda i: (i, 0)),
        out_shape=jax.ShapeDtypeStruct((n, _D), jnp.float32),
        compiler_params=pltpu.CompilerParams(
            dimension_semantics=("arbitrary",),
        ),
    )


def kernel(tokens_l0, tokens_l1, tokens_l2, time_embed, E0, E1, E2, W1, b1, W2, b2):
    b, l, d = time_embed.shape
    n = b * l
    nrows = n // _IDX_W

    t0 = tokens_l0.reshape(nrows, _IDX_W).astype(jnp.int32)
    t1 = tokens_l1.reshape(nrows, _IDX_W).astype(jnp.int32)
    t2 = tokens_l2.reshape(nrows, _IDX_W).astype(jnp.int32)

    h = _sc_gather_sum(nrows)(t0, t1, t2, E0, E1, E2)
    h = h.reshape(n, d)

    x = time_embed.reshape(n, d)
    out = _tc_mlp_add(n, 8192)(
        h, x, W1.T, b1.reshape(1, d), W2.T, b2.reshape(1, d)
    )
    return out.reshape(b, l, d)


# speedup vs baseline: 2.6907x; 2.6907x over previous
import functools
import math

import jax
import jax.numpy as jnp
from jax import lax
from jax.experimental import pallas as pl
from jax.experimental.pallas import tpu as pltpu
from jax.experimental.pallas import tpu_sc as plsc

_D = 64
_IDX_W = 128

_info = plsc.get_sparse_core_info()
_NC, _NS = _info.num_cores, _info.num_subcores
_NW = _NC * _NS


def _sc_gather_sum(nrows: int):
    rows_per_w = nrows // _NW
    mesh = plsc.VectorSubcoreMesh(core_axis_name="c", subcore_axis_name="s")

    @functools.partial(
        pl.kernel,
        mesh=mesh,
        compiler_params=pltpu.CompilerParams(use_tc_tiling_on_sc=False),
        out_type=jax.ShapeDtypeStruct((nrows, _IDX_W, _D), jnp.float32),
        scratch_types=[
            pltpu.VMEM((_IDX_W,), jnp.int32),
            pltpu.VMEM((_IDX_W,), jnp.int32),
            pltpu.VMEM((_IDX_W,), jnp.int32),
            pltpu.VMEM((_IDX_W, _D), jnp.float32),
            pltpu.VMEM((_IDX_W, _D), jnp.float32),
            pltpu.VMEM((_IDX_W, _D), jnp.float32),
            pltpu.SemaphoreType.DMA,
            pltpu.SemaphoreType.DMA,
            pltpu.SemaphoreType.DMA,
        ],
    )
    def k(t0, t1, t2, e0, e1, e2, out, i0, i1, i2, r0, r1, r2, s0, s1, s2):
        wid = lax.axis_index("s") * _NC + lax.axis_index("c")
        row0 = wid * rows_per_w

        def chunk(c, carry):
            row = row0 + c
            pltpu.sync_copy(t0.at[row], i0)
            pltpu.sync_copy(t1.at[row], i1)
            pltpu.sync_copy(t2.at[row], i2)
            cp0 = pltpu.async_copy(e0.at[i0], r0, s0)
            cp1 = pltpu.async_copy(e1.at[i1], r1, s1)
            cp2 = pltpu.async_copy(e2.at[i2], r2, s2)
            cp0.wait()
            cp1.wait()
            cp2.wait()

            def add_row(i, carry2):
                for j in range(_D // 16):
                    sl = pl.ds(j * 16, 16)
                    r0[i, sl] = r0[i, sl] + r1[i, sl] + r2[i, sl]
                return carry2

            lax.fori_loop(0, _IDX_W, add_row, 0)
            pltpu.sync_copy(r0, out.at[row])
            return carry

        lax.fori_loop(0, rows_per_w, chunk, 0)

    return k


def _tc_mlp_add(n: int, bn: int):
    grid = n // bn
    inv_sqrt2 = 1.0 / math.sqrt(2.0)

    def body(h_ref, x_ref, w1_ref, b1_ref, w2_ref, b2_ref, o_ref):
        x = x_ref[...]
        y = jnp.dot(x, w1_ref[...], preferred_element_type=jnp.float32)
        y = y + b1_ref[...]
        y = 0.5 * y * (1.0 + lax.erf(y * inv_sqrt2))
        z = jnp.dot(y, w2_ref[...], preferred_element_type=jnp.float32)
        o_ref[...] = h_ref[...] + z + b2_ref[...]

    return pl.pallas_call(
        body,
        grid=(grid,),
        in_specs=[
            pl.BlockSpec((bn, _D), lambda i: (i, 0)),
            pl.BlockSpec((bn, _D), lambda i: (i, 0)),
            pl.BlockSpec((_D, _D), lambda i: (0, 0)),
            pl.BlockSpec((1, _D), lambda i: (0, 0)),
            pl.BlockSpec((_D, _D), lambda i: (0, 0)),
            pl.BlockSpec((1, _D), lambda i: (0, 0)),
        ],
        out_specs=pl.BlockSpec((bn, _D), lambda i: (i, 0)),
        out_shape=jax.ShapeDtypeStruct((n, _D), jnp.float32),
        compiler_params=pltpu.CompilerParams(
            dimension_semantics=("arbitrary",),
        ),
    )


def kernel(tokens_l0, tokens_l1, tokens_l2, time_embed, E0, E1, E2, W1, b1, W2, b2):
    b, l, d = time_embed.shape
    n = b * l
    nrows = n // _IDX_W

    t0 = tokens_l0.reshape(nrows, _IDX_W).astype(jnp.int32)
    t1 = tokens_l1.reshape(nrows, _IDX_W).astype(jnp.int32)
    t2 = tokens_l2.reshape(nrows, _IDX_W).astype(jnp.int32)

    h = _sc_gather_sum(nrows)(t0, t1, t2, E0, E1, E2)
    h = h.reshape(n, d)

    x = time_embed.reshape(n, d)
    out = _tc_mlp_add(n, 8192)(
        h, x, W1.T, b1.reshape(1, d), W2.T, b2.reshape(1, d)
    )
    return out.reshape(b, l, d)


# speedup vs baseline: 2.6932x; 1.0009x over previous
import functools
import math

import jax
import jax.numpy as jnp
from jax import lax
from jax.experimental import pallas as pl
from jax.experimental.pallas import tpu as pltpu
from jax.experimental.pallas import tpu_sc as plsc

_D = 64
_IDX_W = 128

_info = plsc.get_sparse_core_info()
_NC, _NS = _info.num_cores, _info.num_subcores
_NW = _NC * _NS


def _sc_gather_sum(nrows: int):
    rows_per_w = nrows // _NW
    mesh = plsc.VectorSubcoreMesh(core_axis_name="c", subcore_axis_name="s")

    @functools.partial(
        pl.kernel,
        mesh=mesh,
        compiler_params=pltpu.CompilerParams(use_tc_tiling_on_sc=False),
        out_type=jax.ShapeDtypeStruct((nrows * _IDX_W, 128), jnp.float32),
        scratch_types=[
            pltpu.VMEM((_IDX_W,), jnp.int32),
            pltpu.VMEM((_IDX_W,), jnp.int32),
            pltpu.VMEM((_IDX_W,), jnp.int32),
            pltpu.VMEM((_IDX_W, _D), jnp.float32),
            pltpu.VMEM((_IDX_W, _D), jnp.float32),
            pltpu.VMEM((_IDX_W, _D), jnp.float32),
            pltpu.SemaphoreType.DMA,
            pltpu.SemaphoreType.DMA,
            pltpu.SemaphoreType.DMA,
        ],
    )
    def k(t0, t1, t2, e0, e1, e2, out, i0, i1, i2, r0, r1, r2, s0, s1, s2):
        wid = lax.axis_index("s") * _NC + lax.axis_index("c")
        row0 = wid * rows_per_w

        def chunk(c, carry):
            row = row0 + c
            pltpu.sync_copy(t0.at[row], i0)
            pltpu.sync_copy(t1.at[row], i1)
            pltpu.sync_copy(t2.at[row], i2)
            cp0 = pltpu.async_copy(e0.at[i0], r0, s0)
            cp1 = pltpu.async_copy(e1.at[i1], r1, s1)
            cp2 = pltpu.async_copy(e2.at[i2], r2, s2)
            cp0.wait()
            cp1.wait()
            cp2.wait()

            def add_row(i, carry2):
                for j in range(_D // 16):
                    sl = pl.ds(j * 16, 16)
                    r0[i, sl] = r0[i, sl] + r1[i, sl] + r2[i, sl]
                return carry2

            lax.fori_loop(0, _IDX_W, add_row, 0)
            pltpu.sync_copy(r0, out.at[pl.ds(row * _IDX_W, _IDX_W), pl.ds(0, _D)])
            return carry

        lax.fori_loop(0, rows_per_w, chunk, 0)

    return k


def _tc_mlp_add(b: int, l: int, bb: int):
    grid = b // bb
    bn = bb * l
    inv_sqrt2 = 1.0 / math.sqrt(2.0)

    def body(h_ref, x_ref, w1_ref, b1_ref, w2_ref, b2_ref, o_ref):
        x = x_ref[...].reshape(bn, _D)
        y = jnp.dot(x, w1_ref[...], preferred_element_type=jnp.float32)
        y = y + b1_ref[...]
        y = 0.5 * y * (1.0 + lax.erf(y * inv_sqrt2))
        z = jnp.dot(y, w2_ref[...], preferred_element_type=jnp.float32)
        o = h_ref[:, : _D] + z + b2_ref[...]
        o_ref[...] = o.reshape(bb, l, _D)

    return pl.pallas_call(
        body,
        grid=(grid,),
        in_specs=[
            pl.BlockSpec((bn, 128), lambda i: (i, 0)),
            pl.BlockSpec((bb, l, _D), lambda i: (i, 0, 0)),
            pl.BlockSpec((_D, _D), lambda i: (0, 0)),
            pl.BlockSpec((1, _D), lambda i: (0, 0)),
            pl.BlockSpec((_D, _D), lambda i: (0, 0)),
            pl.BlockSpec((1, _D), lambda i: (0, 0)),
        ],
        out_specs=pl.BlockSpec((bb, l, _D), lambda i: (i, 0, 0)),
        out_shape=jax.ShapeDtypeStruct((b, l, _D), jnp.float32),
        compiler_params=pltpu.CompilerParams(
            dimension_semantics=("arbitrary",),
        ),
    )


def kernel(tokens_l0, tokens_l1, tokens_l2, time_embed, E0, E1, E2, W1, b1, W2, b2):
    b, l, d = time_embed.shape
    n = b * l
    nrows = n // _IDX_W

    t0 = tokens_l0.reshape(nrows, _IDX_W).astype(jnp.int32)
    t1 = tokens_l1.reshape(nrows, _IDX_W).astype(jnp.int32)
    t2 = tokens_l2.reshape(nrows, _IDX_W).astype(jnp.int32)

    h = _sc_gather_sum(nrows)(t0, t1, t2, E0, E1, E2)

    out = _tc_mlp_add(b, l, 8)(
        h, time_embed, W1.T, b1.reshape(1, d), W2.T, b2.reshape(1, d)
    )
    return out


# speedup vs baseline: 3.5174x; 1.3060x over previous
import functools
import math

import jax
import jax.numpy as jnp
from jax import lax
from jax.experimental import pallas as pl
from jax.experimental.pallas import tpu as pltpu
from jax.experimental.pallas import tpu_sc as plsc

_D = 64
_IDX_W = 128

_info = plsc.get_sparse_core_info()
_NC, _NS = _info.num_cores, _info.num_subcores
_NW = _NC * _NS


def _sc_gather_sum(nrows: int):
    rows_per_w = nrows // _NW
    half = rows_per_w // 2
    mesh = plsc.VectorSubcoreMesh(core_axis_name="c", subcore_axis_name="s")

    @functools.partial(
        pl.kernel,
        mesh=mesh,
        compiler_params=pltpu.CompilerParams(use_tc_tiling_on_sc=False),
        out_type=jax.ShapeDtypeStruct((nrows * _IDX_W, 128), jnp.float32),
        scratch_types=[
            pltpu.VMEM((rows_per_w, _IDX_W), jnp.int32),
            pltpu.VMEM((rows_per_w, _IDX_W), jnp.int32),
            pltpu.VMEM((rows_per_w, _IDX_W), jnp.int32),
            pltpu.VMEM((2, _IDX_W, _D), jnp.float32),
            pltpu.VMEM((2, _IDX_W, _D), jnp.float32),
            pltpu.VMEM((2, _IDX_W, _D), jnp.float32),
            pltpu.SemaphoreType.DMA,
            pltpu.SemaphoreType.DMA,
            pltpu.SemaphoreType.DMA,
            pltpu.SemaphoreType.DMA,
            pltpu.SemaphoreType.DMA,
            pltpu.SemaphoreType.DMA,
            pltpu.SemaphoreType.DMA,
            pltpu.SemaphoreType.DMA,
        ],
    )
    def k(t0, t1, t2, e0, e1, e2, out,
          I0, I1, I2, r0, r1, r2,
          g00, g01, g02, g10, g11, g12, w0, w1):
        wid = lax.axis_index("s") * _NC + lax.axis_index("c")
        row0 = wid * rows_per_w
        gsems = ((g00, g01, g02), (g10, g11, g12))
        wsems = (w0, w1)

        pltpu.sync_copy(t0.at[pl.ds(row0, rows_per_w), :], I0)
        pltpu.sync_copy(t1.at[pl.ds(row0, rows_per_w), :], I1)
        pltpu.sync_copy(t2.at[pl.ds(row0, rows_per_w), :], I2)

        def issue(c, s):
            pltpu.async_copy(e0.at[I0.at[c]], r0.at[s], gsems[s][0])
            pltpu.async_copy(e1.at[I1.at[c]], r1.at[s], gsems[s][1])
            pltpu.async_copy(e2.at[I2.at[c]], r2.at[s], gsems[s][2])

        def drain_w(s):
            pltpu.make_async_copy(
                e0.at[pl.ds(0, _IDX_W)], r0.at[s], wsems[s]).wait()

        def complete(c, s):
            for m in range(3):
                pltpu.make_async_copy(
                    e0.at[pl.ds(0, _IDX_W)], r0.at[s], gsems[s][m]).wait()

            def add_row(i, carry):
                for j in range(_D // 16):
                    sl = pl.ds(j * 16, 16)
                    plsc.addupdate(r0.at[s, i, sl], r1[s, i, sl])
                    plsc.addupdate(r0.at[s, i, sl], r2[s, i, sl])
                return carry

            lax.fori_loop(0, _IDX_W, add_row, 0)
            pltpu.async_copy(
                r0.at[s],
                out.at[pl.ds((row0 + c) * _IDX_W, _IDX_W), pl.ds(0, _D)],
                wsems[s])

        issue(0, 0)

        def body(g, carry):
            c1 = 2 * g + 1

            @pl.when(g >= 1)
            def _():
                drain_w(1)

            issue(c1, 1)
            complete(2 * g, 0)
            c2 = 2 * g + 2

            @pl.when(c2 < rows_per_w)
            def _():
                drain_w(0)
                issue(c2, 0)

            complete(c1, 1)
            return carry

        lax.fori_loop(0, half, body, 0)
        drain_w(0)
        drain_w(1)

    return k


def _tc_mlp_add(b: int, l: int, bb: int):
    grid = b // bb
    bn = bb * l
    inv_sqrt2 = 1.0 / math.sqrt(2.0)

    def body(h_ref, x_ref, w1_ref, b1_ref, w2_ref, b2_ref, o_ref):
        x = x_ref[...].reshape(bn, _D)
        y = jnp.dot(x, w1_ref[...], preferred_element_type=jnp.float32)
        y = y + b1_ref[...]
        y = 0.5 * y * (1.0 + lax.erf(y * inv_sqrt2))
        z = jnp.dot(y, w2_ref[...], preferred_element_type=jnp.float32)
        o = h_ref[:, : _D] + z + b2_ref[...]
        o_ref[...] = o.reshape(bb, l, _D)

    return pl.pallas_call(
        body,
        grid=(grid,),
        in_specs=[
            pl.BlockSpec((bn, 128), lambda i: (i, 0)),
            pl.BlockSpec((bb, l, _D), lambda i: (i, 0, 0)),
            pl.BlockSpec((_D, _D), lambda i: (0, 0)),
            pl.BlockSpec((1, _D), lambda i: (0, 0)),
            pl.BlockSpec((_D, _D), lambda i: (0, 0)),
            pl.BlockSpec((1, _D), lambda i: (0, 0)),
        ],
        out_specs=pl.BlockSpec((bb, l, _D), lambda i: (i, 0, 0)),
        out_shape=jax.ShapeDtypeStruct((b, l, _D), jnp.float32),
        compiler_params=pltpu.CompilerParams(
            dimension_semantics=("arbitrary",),
        ),
    )


def kernel(tokens_l0, tokens_l1, tokens_l2, time_embed, E0, E1, E2, W1, b1, W2, b2):
    b, l, d = time_embed.shape
    n = b * l
    nrows = n // _IDX_W

    t0 = tokens_l0.reshape(nrows, _IDX_W).astype(jnp.int32)
    t1 = tokens_l1.reshape(nrows, _IDX_W).astype(jnp.int32)
    t2 = tokens_l2.reshape(nrows, _IDX_W).astype(jnp.int32)

    h = _sc_gather_sum(nrows)(t0, t1, t2, E0, E1, E2)

    out = _tc_mlp_add(b, l, 64)(
        h, time_embed, W1.T, b1.reshape(1, d), W2.T, b2.reshape(1, d)
    )
    return out
